# baseline (speedup 1.0000x reference)
import functools

import jax
import jax.numpy as jnp
from jax import lax
from jax.experimental import pallas as pl
from jax.experimental.pallas import tpu as pltpu
from jax.experimental.pallas import tpu_sc as plsc

_NW = 32
_WIN = 512
_CAP = 896
_D = 64


def _make_scan_kernel(B, N):
    n_win = (N + _WIN - 1) // _WIN
    n_win_w = (n_win + _NW - 1) // _NW
    idx_ch = 4096
    mesh = plsc.VectorSubcoreMesh(core_axis_name="c", subcore_axis_name="s")

    @functools.partial(
        pl.kernel,
        mesh=mesh,
        out_type=(
            jax.ShapeDtypeStruct((_NW, _D * _CAP), jnp.float32),
            jax.ShapeDtypeStruct((_NW, _CAP), jnp.int32),
        ),
        scratch_types=[
            pltpu.VMEM((idx_ch,), jnp.int32),
            pltpu.VMEM((_CAP,), jnp.int32),
            pltpu.VMEM((_CAP,), jnp.int32),
            pltpu.VMEM((8, 8, _WIN), jnp.float32),
            pltpu.VMEM((8, 8, _WIN), jnp.float32),
            pltpu.VMEM((_D * _CAP,), jnp.float32),
            pltpu.VMEM((_CAP,), jnp.int32),
            pltpu.VMEM((32,), jnp.int32),
            pltpu.VMEM((32,), jnp.int32),
            pltpu.SemaphoreType.DMA,
            pltpu.SemaphoreType.DMA,
        ],
        compiler_params=pltpu.CompilerParams(needs_layout_passes=False),
    )
    def k(e1_hbm, et3_hbm, stage_hbm, jout_hbm,
          idx_v, sel_i, sel_j, win0_v, win1_v, stage_v, jbuf_v,
          pend_i, pend_j, sem0, sem1):
        wid = lax.axis_index("s") * 2 + lax.axis_index("c")
        w_start = lax.shift_right_logical(wid * n_win, 5)
        w_end = lax.shift_right_logical((wid + 1) * n_win, 5)
        nw = w_end - w_start
        lo_val = w_start * _WIN
        hi_val = lax.min(w_end * _WIN, N)

        for m in range(_CAP // 16):
            jbuf_v[pl.ds(m * 16, 16)] = jnp.full((16,), -1, jnp.int32)

        lanes = lax.broadcasted_iota(jnp.int32, (16,), 0)

        def presel_chunk(ch):
            pltpu.sync_copy(e1_hbm.at[pl.ds(ch * idx_ch, idx_ch)], idx_v)

            def presel(g, n_sel):
                v = idx_v[pl.ds(g * 16, 16)]
                m = (v >= lo_val) & (v < hi_val) & (n_sel <= _CAP - 16)
                cnt = plsc.all_reduce_population_count(m)[0]
                plsc.store_compressed(sel_i.at[pl.ds(n_sel, 16)], v, mask=m)
                plsc.store_compressed(
                    sel_j.at[pl.ds(n_sel, 16)],
                    ch * idx_ch + g * 16 + lanes, mask=m)
                return n_sel + cnt

            return presel

        n_sel = 0
        for ch in range(B // idx_ch):
            n_sel = lax.fori_loop(0, idx_ch // 16, presel_chunk(ch), n_sel)
        n_vreg = lax.shift_right_logical(n_sel + 15, 4)

        def widx(t):
            return w_start + lax.min(t, nw - 1)

        def dma_start_of(w):
            return pl.multiple_of(
                lax.min(w * _WIN, ((N - _WIN) // 128) * 128 + 128), 128)

        def fire(t, win_v, sem):
            return pltpu.async_copy(
                et3_hbm.at[:, :, pl.ds(dma_start_of(widx(t)), _WIN)],
                win_v, sem)

        def drain(win_v, sem):
            pltpu.make_async_copy(
                et3_hbm.at[:, :, pl.ds(0, _WIN)], win_v, sem).wait()

        def flush(win_v, dstart, off, valid_n):
            ok = lax.min(valid_n, _CAP - off)
            fmask = lanes < ok
            pv = pend_i[pl.ds(0, 16)]
            pj = pend_j[pl.ds(0, 16)]
            vloc = pv - dstart
            for c in range(_D):
                g16 = plsc.load_gather(
                    win_v,
                    [jnp.full((16,), c >> 3, jnp.int32),
                     jnp.full((16,), c & 7, jnp.int32),
                     vloc],
                    mask=fmask)
                plsc.store_compressed(
                    stage_v.at[pl.ds(c * _CAP + off, 16)], g16, mask=fmask)
            plsc.store_compressed(jbuf_v.at[pl.ds(off, 16)], pj, mask=fmask)

        def process(w, win_v, off):
            wlo = w * _WIN
            whi = lax.min(wlo + _WIN, N)
            dstart = dma_start_of(w)

            def visit(m_, carry_):
                off_, np_ = carry_
                v = sel_i[pl.ds(m_ * 16, 16)]
                jv = sel_j[pl.ds(m_ * 16, 16)]
                in_rng = (m_ * 16 + lanes) < n_sel
                hit = (v >= wlo) & (v < whi) & in_rng & (np_ <= 16)
                cnt = plsc.all_reduce_population_count(hit)[0]

                @pl.when(cnt > 0)
                def _():
                    plsc.store_compressed(
                        pend_i.at[pl.ds(np_, 16)], v, mask=hit)
                    plsc.store_compressed(
                        pend_j.at[pl.ds(np_, 16)], jv, mask=hit)

                do_flush = (np_ + cnt >= 16) & (off_ <= _CAP - 16)

                @pl.when(do_flush)
                def _():
                    flush(win_v, dstart, off_, 16)
                    rem_i = pend_i[pl.ds(16, 16)]
                    rem_j = pend_j[pl.ds(16, 16)]
                    pend_i[pl.ds(0, 16)] = rem_i
                    pend_j[pl.ds(0, 16)] = rem_j

                adv = lax.select(do_flush, 16, 0)
                return (off_ + adv, np_ + cnt - adv)

            def visit4(p_, carry_):
                for u in range(4):
                    carry_ = visit(4 * p_ + u, carry_)
                return carry_

            off, np_ = lax.fori_loop(
                0, lax.shift_right_logical(n_vreg + 3, 2), visit4, (off, 0))

            can = (np_ > 0) & (off <= _CAP - 16)

            @pl.when(can)
            def _():
                flush(win_v, dstart, off, np_)

            return off + lax.select(can, np_, 0)

        fire(0, win0_v, sem0)
        fire(1, win1_v, sem1)

        def pair(p, off):
            drain(win0_v, sem0)
            off = process(widx(2 * p), win0_v, off)
            fire(2 * p + 2, win0_v, sem0)
            drain(win1_v, sem1)
            off = process(widx(2 * p + 1), win1_v, off)
            fire(2 * p + 3, win1_v, sem1)
            return off

        lax.fori_loop(0, (n_win_w + 1) // 2, pair, 0)
        drain(win0_v, sem0)
        drain(win1_v, sem1)

        pltpu.sync_copy(stage_v, stage_hbm.at[wid])
        pltpu.sync_copy(jbuf_v, jout_hbm.at[wid])

    return k


def _make_scatter_kernel(B, NR):
    b_per_w = B // _NW
    mesh = plsc.VectorSubcoreMesh(core_axis_name="c", subcore_axis_name="s")

    @functools.partial(
        pl.kernel,
        mesh=mesh,
        out_type=(
            jax.ShapeDtypeStruct((B, _D), jnp.float32),
            jax.ShapeDtypeStruct((B, _D), jnp.float32),
        ),
        scratch_types=[
            pltpu.VMEM((_CAP, _D), jnp.float32),
            pltpu.VMEM((128,), jnp.int32),
            pltpu.VMEM((b_per_w,), jnp.int32),
            pltpu.VMEM((b_per_w, _D), jnp.float32),
            pltpu.SemaphoreType.DMA,
            pltpu.SemaphoreType.DMA,
        ],
        compiler_params=pltpu.CompilerParams(use_tc_tiling_on_sc=False),
    )
    def k(rows_hbm, jout_hbm, q_hbm, emb_r_hbm, out_h_hbm, out_q_hbm,
          rows_v, jv_v, qidx_v, qrows_v, sem_s, sem_g):
        wid = lax.axis_index("s") * 2 + lax.axis_index("c")
        base = wid * b_per_w

        pltpu.sync_copy(q_hbm.at[pl.ds(base, b_per_w)], qidx_v)
        gathers = []
        for c in range(b_per_w // 128):
            s = pl.ds(c * 128, 128)
            gathers.append(pltpu.async_copy(
                emb_r_hbm.at[qidx_v.at[s]], qrows_v.at[s], sem_g))

        pltpu.sync_copy(rows_hbm.at[wid], rows_v)
        scatters = []
        for c in range(_CAP // 128):
            pltpu.sync_copy(jout_hbm.at[wid, pl.ds(c * 128, 128)], jv_v)
            scatters.append(pltpu.async_copy(
                rows_v.at[pl.ds(c * 128, 128)],
                out_h_hbm.at[plsc.Indices(jv_v, ignored_value=-1)],
                sem_s))
            scatters[-1].wait()

        for cp in gathers:
            cp.wait()
        pltpu.sync_copy(qrows_v, out_q_hbm.at[pl.ds(base, b_per_w)])

    return k


def _gather2(batch_e1, batch_q, emb_e, emb_r):
    B = batch_e1.shape[0]
    N = emb_e.shape[0]
    et3 = emb_e.T.reshape(8, 8, N)
    k1 = _make_scan_kernel(B, N)
    stage, jout = k1(batch_e1, et3)
    rows = jnp.swapaxes(stage.reshape(_NW, _D, _CAP), 1, 2)
    k2 = _make_scatter_kernel(B, emb_r.shape[0])
    out_h, out_q = k2(rows, jout, batch_q, emb_r)
    return out_h, out_q


def kernel(batch_e1, batch_q, emb_e, emb_r):
    if batch_e1.dtype != jnp.int32:
        batch_e1 = batch_e1.astype(jnp.int32)
        batch_q = batch_q.astype(jnp.int32)
    return _gather2(batch_e1, batch_q, emb_e, emb_r)


# speedup vs baseline: 1.0693x; 1.0693x over previous
import functools

import jax
import jax.numpy as jnp
from jax import lax
from jax.experimental import pallas as pl
from jax.experimental.pallas import tpu as pltpu
from jax.experimental.pallas import tpu_sc as plsc

_NW = 32
_WIN = 512
_CAP = 896
_D = 64


def _make_scan_kernel(B, N):
    n_win = (N + _WIN - 1) // _WIN
    n_win_w = (n_win + _NW - 1) // _NW
    idx_ch = 4096
    mesh = plsc.VectorSubcoreMesh(core_axis_name="c", subcore_axis_name="s")

    @functools.partial(
        pl.kernel,
        mesh=mesh,
        out_type=(
            jax.ShapeDtypeStruct((_NW, _D * _CAP), jnp.float32),
            jax.ShapeDtypeStruct((_NW, _CAP), jnp.int32),
        ),
        scratch_types=[
            pltpu.VMEM((idx_ch,), jnp.int32),
            pltpu.VMEM((_CAP,), jnp.int32),
            pltpu.VMEM((_CAP,), jnp.int32),
            pltpu.VMEM((8, 8, _WIN), jnp.float32),
            pltpu.VMEM((8, 8, _WIN), jnp.float32),
            pltpu.VMEM((_D * _CAP,), jnp.float32),
            pltpu.VMEM((_CAP,), jnp.int32),
            pltpu.VMEM((32,), jnp.int32),
            pltpu.VMEM((32,), jnp.int32),
            pltpu.SemaphoreType.DMA,
            pltpu.SemaphoreType.DMA,
        ],
        compiler_params=pltpu.CompilerParams(needs_layout_passes=False),
    )
    def k(e1_hbm, et3_hbm, stage_hbm, jout_hbm,
          idx_v, sel_i, sel_j, win0_v, win1_v, stage_v, jbuf_v,
          pend_i, pend_j, sem0, sem1):
        wid = lax.axis_index("s") * 2 + lax.axis_index("c")
        w_start = lax.shift_right_logical(wid * n_win, 5)
        w_end = lax.shift_right_logical((wid + 1) * n_win, 5)
        nw = w_end - w_start
        lo_val = w_start * _WIN
        hi_val = lax.min(w_end * _WIN, N)

        for m in range(_CAP // 16):
            jbuf_v[pl.ds(m * 16, 16)] = jnp.full((16,), -1, jnp.int32)

        lanes = lax.broadcasted_iota(jnp.int32, (16,), 0)

        def presel_chunk(ch):
            pltpu.sync_copy(e1_hbm.at[pl.ds(ch * idx_ch, idx_ch)], idx_v)

            def presel(g, n_sel):
                v = idx_v[pl.ds(g * 16, 16)]
                m = (v >= lo_val) & (v < hi_val) & (n_sel <= _CAP - 16)
                cnt = plsc.all_reduce_population_count(m)[0]
                plsc.store_compressed(sel_i.at[pl.ds(n_sel, 16)], v, mask=m)
                plsc.store_compressed(
                    sel_j.at[pl.ds(n_sel, 16)],
                    ch * idx_ch + g * 16 + lanes, mask=m)
                return n_sel + cnt

            return presel

        n_sel = 0
        for ch in range(B // idx_ch):
            n_sel = lax.fori_loop(0, idx_ch // 16, presel_chunk(ch), n_sel)
        n_vreg = lax.shift_right_logical(n_sel + 15, 4)

        def widx(t):
            return w_start + lax.min(t, nw - 1)

        def dma_start_of(w):
            return pl.multiple_of(
                lax.min(w * _WIN, ((N - _WIN) // 128) * 128 + 128), 128)

        def fire(t, win_v, sem):
            return pltpu.async_copy(
                et3_hbm.at[:, :, pl.ds(dma_start_of(widx(t)), _WIN)],
                win_v, sem)

        def drain(win_v, sem):
            pltpu.make_async_copy(
                et3_hbm.at[:, :, pl.ds(0, _WIN)], win_v, sem).wait()

        def flush(win_v, dstart, off, valid_n):
            ok = lax.min(valid_n, _CAP - off)
            fmask = lanes < ok
            pv = pend_i[pl.ds(0, 16)]
            pj = pend_j[pl.ds(0, 16)]
            vloc = pv - dstart
            for c in range(_D):
                g16 = plsc.load_gather(
                    win_v,
                    [jnp.full((16,), c >> 3, jnp.int32),
                     jnp.full((16,), c & 7, jnp.int32),
                     vloc],
                    mask=fmask)
                plsc.store_compressed(
                    stage_v.at[pl.ds(c * _CAP + off, 16)], g16, mask=fmask)
            plsc.store_compressed(jbuf_v.at[pl.ds(off, 16)], pj, mask=fmask)

        def process(w, win_v, off):
            wlo = w * _WIN
            whi = lax.min(wlo + _WIN, N)
            dstart = dma_start_of(w)

            def visit(m_, carry_):
                off_, np_ = carry_
                v = sel_i[pl.ds(m_ * 16, 16)]
                jv = sel_j[pl.ds(m_ * 16, 16)]
                in_rng = (m_ * 16 + lanes) < n_sel
                hit = (v >= wlo) & (v < whi) & in_rng & (np_ <= 16)
                cnt = plsc.all_reduce_population_count(hit)[0]

                @pl.when(cnt > 0)
                def _():
                    plsc.store_compressed(
                        pend_i.at[pl.ds(np_, 16)], v, mask=hit)
                    plsc.store_compressed(
                        pend_j.at[pl.ds(np_, 16)], jv, mask=hit)

                do_flush = (np_ + cnt >= 16) & (off_ <= _CAP - 16)

                @pl.when(do_flush)
                def _():
                    flush(win_v, dstart, off_, 16)
                    rem_i = pend_i[pl.ds(16, 16)]
                    rem_j = pend_j[pl.ds(16, 16)]
                    pend_i[pl.ds(0, 16)] = rem_i
                    pend_j[pl.ds(0, 16)] = rem_j

                adv = lax.select(do_flush, 16, 0)
                return (off_ + adv, np_ + cnt - adv)

            def visit2(p_, carry_):
                carry_ = visit(2 * p_, carry_)
                return visit(2 * p_ + 1, carry_)

            off, np_ = lax.fori_loop(
                0, lax.shift_right_logical(n_vreg + 1, 1), visit2, (off, 0))

            can = (np_ > 0) & (off <= _CAP - 16)

            @pl.when(can)
            def _():
                flush(win_v, dstart, off, np_)

            return off + lax.select(can, np_, 0)

        fire(0, win0_v, sem0)
        fire(1, win1_v, sem1)

        def pair(p, off):
            drain(win0_v, sem0)
            off = process(widx(2 * p), win0_v, off)
            fire(2 * p + 2, win0_v, sem0)
            drain(win1_v, sem1)
            off = process(widx(2 * p + 1), win1_v, off)
            fire(2 * p + 3, win1_v, sem1)
            return off

        lax.fori_loop(0, (n_win_w + 1) // 2, pair, 0)
        drain(win0_v, sem0)
        drain(win1_v, sem1)

        pltpu.sync_copy(stage_v, stage_hbm.at[wid])
        pltpu.sync_copy(jbuf_v, jout_hbm.at[wid])

    return k


def _make_scatter_kernel(B, NR):
    b_per_w = B // _NW
    mesh = plsc.VectorSubcoreMesh(core_axis_name="c", subcore_axis_name="s")

    @functools.partial(
        pl.kernel,
        mesh=mesh,
        out_type=(
            jax.ShapeDtypeStruct((B, _D), jnp.float32),
            jax.ShapeDtypeStruct((B, _D), jnp.float32),
        ),
        scratch_types=[
            pltpu.VMEM((_CAP, _D), jnp.float32),
            pltpu.VMEM((128,), jnp.int32),
            pltpu.VMEM((b_per_w,), jnp.int32),
            pltpu.VMEM((b_per_w, _D), jnp.float32),
            pltpu.SemaphoreType.DMA,
            pltpu.SemaphoreType.DMA,
        ],
        compiler_params=pltpu.CompilerParams(use_tc_tiling_on_sc=False),
    )
    def k(rows_hbm, jout_hbm, q_hbm, emb_r_hbm, out_h_hbm, out_q_hbm,
          rows_v, jv_v, qidx_v, qrows_v, sem_s, sem_g):
        wid = lax.axis_index("s") * 2 + lax.axis_index("c")
        base = wid * b_per_w

        pltpu.sync_copy(q_hbm.at[pl.ds(base, b_per_w)], qidx_v)
        gathers = []
        for c in range(b_per_w // 128):
            s = pl.ds(c * 128, 128)
            gathers.append(pltpu.async_copy(
                emb_r_hbm.at[qidx_v.at[s]], qrows_v.at[s], sem_g))

        pltpu.sync_copy(rows_hbm.at[wid], rows_v)
        scatters = []
        for c in range(_CAP // 128):
            pltpu.sync_copy(jout_hbm.at[wid, pl.ds(c * 128, 128)], jv_v)
            scatters.append(pltpu.async_copy(
                rows_v.at[pl.ds(c * 128, 128)],
                out_h_hbm.at[plsc.Indices(jv_v, ignored_value=-1)],
                sem_s))
            scatters[-1].wait()

        for cp in gathers:
            cp.wait()
        pltpu.sync_copy(qrows_v, out_q_hbm.at[pl.ds(base, b_per_w)])

    return k


def _gather2(batch_e1, batch_q, emb_e, emb_r):
    B = batch_e1.shape[0]
    N = emb_e.shape[0]
    et3 = emb_e.T.reshape(8, 8, N)
    k1 = _make_scan_kernel(B, N)
    stage, jout = k1(batch_e1, et3)
    rows = jnp.swapaxes(stage.reshape(_NW, _D, _CAP), 1, 2)
    k2 = _make_scatter_kernel(B, emb_r.shape[0])
    out_h, out_q = k2(rows, jout, batch_q, emb_r)
    return out_h, out_q


def kernel(batch_e1, batch_q, emb_e, emb_r):
    if batch_e1.dtype != jnp.int32:
        batch_e1 = batch_e1.astype(jnp.int32)
        batch_q = batch_q.astype(jnp.int32)
    return _gather2(batch_e1, batch_q, emb_e, emb_r)
